# baseline (speedup 1.0000x reference)
import functools

import jax
import jax.numpy as jnp
from jax import lax
from jax.experimental import pallas as pl
from jax.experimental.pallas import tpu as pltpu
from jax.experimental.pallas import tpu_sc as plsc


@functools.lru_cache(maxsize=None)
def _build(B, N, V, D, P):
    BT = B * N
    info = plsc.get_sparse_core_info()
    NC, NS, L = info.num_cores, info.num_subcores, info.num_lanes
    NW = NC * NS
    assert BT % NW == 0
    tok_w = BT // NW
    CH = 2 * N
    assert tok_w % CH == 0
    steps = tok_w // CH
    GC = 128

    mesh = plsc.VectorSubcoreMesh(core_axis_name="c", subcore_axis_name="s")

    @functools.partial(
        pl.kernel,
        mesh=mesh,
        compiler_params=pltpu.CompilerParams(use_tc_tiling_on_sc=False),
        out_type=jax.ShapeDtypeStruct((BT, D), jnp.float32),
        scratch_types=[
            pltpu.VMEM((tok_w,), jnp.int32),
            pltpu.VMEM((CH, D), jnp.float32),
            pltpu.VMEM((N, D), jnp.float32),
            pltpu.SemaphoreType.DMA,
        ],
    )
    def kern(x_hbm, tab_hbm, pos_hbm, out_hbm, idx_v, rows_v, pos_v, sem):
        wid = lax.axis_index("s") * NC + lax.axis_index("c")
        base_w = wid * tok_w
        pltpu.sync_copy(x_hbm.at[pl.ds(base_w, tok_w)], idx_v)
        pltpu.sync_copy(pos_hbm.at[pl.ds(0, N)], pos_v)

        def step(s, carry):
            t0 = s * CH
            copies = []
            for k in range(0, CH, GC):
                sz = min(GC, CH - k)
                copies.append(pltpu.async_copy(
                    tab_hbm.at[idx_v.at[pl.ds(t0 + k, sz)]],
                    rows_v.at[pl.ds(k, sz)],
                    sem))
            for cp in copies:
                cp.wait()

            def add_body(t, c2):
                n = lax.rem(t, N)
                for c in range(D // L):
                    sl = pl.ds(c * L, L)
                    rows_v[t, sl] = rows_v[t, sl] + pos_v[n, sl]
                return c2
            lax.fori_loop(0, CH, add_body, 0)

            pltpu.sync_copy(rows_v, out_hbm.at[pl.ds(base_w + t0, CH)])
            return carry
        lax.fori_loop(0, steps, step, 0)

    return kern


def kernel(x, emb_table, pos_table):
    B, N = x.shape
    V, D = emb_table.shape
    kern = _build(B, N, V, D, pos_table.shape[0])
    out = kern(x.reshape(-1).astype(jnp.int32), emb_table, pos_table)
    return out.reshape(B, N, D)


# speedup vs baseline: 1.4031x; 1.4031x over previous
import functools

import jax
import jax.numpy as jnp
from jax import lax
from jax.experimental import pallas as pl
from jax.experimental.pallas import tpu as pltpu
from jax.experimental.pallas import tpu_sc as plsc

_NBUF = 4


@functools.lru_cache(maxsize=None)
def _build(B, N, V, D, P):
    info = plsc.get_sparse_core_info()
    NC, NS, L = info.num_cores, info.num_subcores, info.num_lanes
    NW = NC * NS
    assert B % NW == 0
    CB = B // NW
    assert CB <= 128
    assert D % L == 0
    NG = D // L
    assert N % 2 == 0

    mesh = plsc.VectorSubcoreMesh(core_axis_name="c", subcore_axis_name="s")

    @functools.partial(
        pl.kernel,
        mesh=mesh,
        compiler_params=pltpu.CompilerParams(use_tc_tiling_on_sc=False),
        out_type=jax.ShapeDtypeStruct((N, B, D), jnp.float32),
        scratch_types=[
            pltpu.VMEM((N, CB), jnp.int32),
            pltpu.VMEM((_NBUF, CB, D), jnp.float32),
            pltpu.VMEM((N, D), jnp.float32),
            [pltpu.SemaphoreType.DMA] * _NBUF,
            [pltpu.SemaphoreType.DMA] * _NBUF,
        ],
    )
    def kern(xt_hbm, tab_hbm, pos_hbm, out_hbm, idx_v, buf_v, pos_v,
             gsems, ssems):
        wid = lax.axis_index("s") * NC + lax.axis_index("c")
        b0 = wid * CB
        pltpu.sync_copy(xt_hbm.at[:, pl.ds(b0, CB)], idx_v)
        pltpu.sync_copy(pos_hbm.at[pl.ds(0, N)], pos_v)

        def gather(n, k):
            return pltpu.async_copy(
                tab_hbm.at[idx_v.at[n]], buf_v.at[k], gsems[k])

        def store(n, k):
            return pltpu.async_copy(
                buf_v.at[k], out_hbm.at[n, pl.ds(b0, CB)], ssems[k])

        for k in range(2):
            gather(k, k)

        def outer(j, carry):
            for k in range(_NBUF):
                n = j * _NBUF + k
                pltpu.make_async_copy(
                    tab_hbm.at[idx_v.at[n]], buf_v.at[k], gsems[k]).wait()
                pvecs = [pos_v[n, pl.ds(c * L, L)] for c in range(NG)]

                def add_body(r, c2):
                    for c in range(NG):
                        sl = pl.ds(c * L, L)
                        buf_v[k, r, sl] = buf_v[k, r, sl] + pvecs[c]
                    return c2
                lax.fori_loop(0, CB, add_body, 0)
                store(n, k)
                km2 = (k - 2) % _NBUF

                @pl.when(n >= 2)
                def _():
                    pltpu.make_async_copy(
                        buf_v.at[km2], out_hbm.at[n - 2, pl.ds(b0, CB)],
                        ssems[km2]).wait()

                @pl.when(n + 2 < N)
                def _():
                    gather(n + 2, (k + 2) % _NBUF)
            return carry
        lax.fori_loop(0, N // _NBUF, outer, 0)
        for n in range(N - 2, N):
            k = n % _NBUF
            pltpu.make_async_copy(
                buf_v.at[k], out_hbm.at[n, pl.ds(b0, CB)], ssems[k]).wait()

    return kern


def kernel(x, emb_table, pos_table):
    B, N = x.shape
    V, D = emb_table.shape
    kern = _build(B, N, V, D, pos_table.shape[0])
    xt = jnp.swapaxes(x, 0, 1).astype(jnp.int32)
    out = kern(xt, emb_table, pos_table)
    return jnp.swapaxes(out, 0, 1)
